# baseline (speedup 1.0000x reference)
import functools

import numpy as np
import jax
import jax.numpy as jnp
from jax import lax
from jax.experimental import pallas as pl
from jax.experimental.pallas import tpu as pltpu

_EDGES = [[0, 1], [1, 2], [2, 3], [3, 4], [0, 5], [5, 6], [6, 7], [7, 8],
          [0, 9], [9, 10], [10, 11], [11, 12], [12, 13], [11, 14], [14, 15],
          [15, 16], [16, 17], [11, 18], [18, 19], [19, 20], [20, 21]]
_DIST = 1
_CH_IN = 3
_CH_OUT = 24


def _neighbor_list(edges, d):
    n = len(edges)
    inf = 1e9
    mat = np.full((n, n), inf)
    for i, a in enumerate(edges):
        for j, b in enumerate(edges):
            if set(a) & set(b):
                mat[i, j] = 1.0
    np.fill_diagonal(mat, 0.0)
    for k in range(n):
        mat = np.minimum(mat, mat[:, k:k + 1] + mat[k:k + 1, :])
    nbl = [[j for j in range(n) if mat[i, j] <= d] for i in range(n)]
    root_nb = list(nbl[0])
    for i in root_nb:
        nbl[i].append(n)
    root_nb.append(n)
    nbl.append(root_nb)
    return nbl


def _mask_np(nbl, cpj_in, cpj_out):
    n = len(nbl)
    mask = np.zeros((n * cpj_out, n * cpj_in), dtype=np.float32)
    for i, nbs in enumerate(nbl):
        for j in nbs:
            mask[i * cpj_out:(i + 1) * cpj_out, j * cpj_in:(j + 1) * cpj_in] = 1.0
    return mask


def _pool_np(edges, cpe):
    n_joints = max(max(e) for e in edges) + 1
    degree = [0] * (n_joints + 1)
    for a, b in edges:
        degree[a] += 1
        degree[b] += 1
    seq_list = []

    def find_seq(j, seq):
        if degree[j] > 2 and j != 0:
            seq_list.append(seq)
            seq = []
        if degree[j] == 1:
            seq_list.append(seq)
            return
        for idx, e in enumerate(edges):
            if e[0] == j:
                find_seq(e[1], seq + [idx])

    find_seq(0, [])
    pooling_list = []
    for seq in seq_list:
        if len(seq) % 2 == 1:
            pooling_list.append([seq[0]])
            seq = seq[1:]
        for i in range(0, len(seq), 2):
            pooling_list.append([seq[i], seq[i + 1]])
    edge_num = len(edges) + 1
    pooling_list.append([edge_num - 1])
    w = np.zeros((len(pooling_list) * cpe, edge_num * cpe), dtype=np.float32)
    for i, pair in enumerate(pooling_list):
        for j in pair:
            w[i * cpe:(i + 1) * cpe, j * cpe:(j + 1) * cpe] += (
                np.eye(cpe, dtype=np.float32) / len(pair))
    return w


_NBL = _neighbor_list(_EDGES, _DIST)
_MASK = _mask_np(_NBL, _CH_IN, _CH_OUT)
_POOL = _pool_np(_EDGES, _CH_OUT)


def _enc_kernel(xt_ref, wt_ref, bias_ref, maskt_ref, pool_ref,
                out_ref, wc_ref, bc_ref):
    @pl.when(pl.program_id(0) == 0)
    def _fold():
        wmt = wt_ref[...] * maskt_ref[...]
        wc_ref[...] = lax.dot_general(
            pool_ref[...], wmt, (((1,), (1,)), ((), ())),
            preferred_element_type=jnp.float32)
        bc_ref[...] = lax.dot_general(
            pool_ref[...], bias_ref[...], (((1,), (0,)), ((), ())),
            preferred_element_type=jnp.float32)

    acc = lax.dot_general(
        wc_ref[...].astype(jnp.bfloat16), xt_ref[...].astype(jnp.bfloat16),
        (((1,), (0,)), ((), ())),
        preferred_element_type=jnp.float32)
    out_ref[...] = jnp.maximum(acc + bc_ref[...], 0.0)


@functools.partial(jax.jit, static_argnames=())
def kernel(input, weight, bias):
    B, in_ch = input.shape
    out_ch = _POOL.shape[0]
    maskt = jnp.asarray(_MASK.T)
    pool = jnp.asarray(_POOL)
    bias2 = bias.reshape(-1, 1)

    n_mask = _MASK.shape[0]

    TB = 16384
    grid = (B // TB,)
    out_t = pl.pallas_call(
        _enc_kernel,
        grid=grid,
        in_specs=[
            pl.BlockSpec((in_ch, TB), lambda i: (0, i)),
            pl.BlockSpec((in_ch, n_mask), lambda i: (0, 0)),
            pl.BlockSpec((n_mask, 1), lambda i: (0, 0)),
            pl.BlockSpec((in_ch, n_mask), lambda i: (0, 0)),
            pl.BlockSpec((out_ch, n_mask), lambda i: (0, 0)),
        ],
        out_specs=pl.BlockSpec((out_ch, TB), lambda i: (0, i)),
        out_shape=jax.ShapeDtypeStruct((out_ch, B), jnp.float32),
        scratch_shapes=[
            pltpu.VMEM((out_ch, in_ch), jnp.float32),
            pltpu.VMEM((out_ch, 1), jnp.float32),
        ],
    )(input.T, weight.T, bias2, maskt, pool)

    return out_t.T[..., None]


# speedup vs baseline: 1.0688x; 1.0688x over previous
import functools

import numpy as np
import jax
import jax.numpy as jnp
from jax import lax
from jax.experimental import pallas as pl
from jax.experimental.pallas import tpu as pltpu

_EDGES = [[0, 1], [1, 2], [2, 3], [3, 4], [0, 5], [5, 6], [6, 7], [7, 8],
          [0, 9], [9, 10], [10, 11], [11, 12], [12, 13], [11, 14], [14, 15],
          [15, 16], [16, 17], [11, 18], [18, 19], [19, 20], [20, 21]]
_DIST = 1
_CH_IN = 3
_CH_OUT = 24


def _neighbor_list(edges, d):
    n = len(edges)
    inf = 1e9
    mat = np.full((n, n), inf)
    for i, a in enumerate(edges):
        for j, b in enumerate(edges):
            if set(a) & set(b):
                mat[i, j] = 1.0
    np.fill_diagonal(mat, 0.0)
    for k in range(n):
        mat = np.minimum(mat, mat[:, k:k + 1] + mat[k:k + 1, :])
    nbl = [[j for j in range(n) if mat[i, j] <= d] for i in range(n)]
    root_nb = list(nbl[0])
    for i in root_nb:
        nbl[i].append(n)
    root_nb.append(n)
    nbl.append(root_nb)
    return nbl


def _mask_np(nbl, cpj_in, cpj_out):
    n = len(nbl)
    mask = np.zeros((n * cpj_out, n * cpj_in), dtype=np.float32)
    for i, nbs in enumerate(nbl):
        for j in nbs:
            mask[i * cpj_out:(i + 1) * cpj_out, j * cpj_in:(j + 1) * cpj_in] = 1.0
    return mask


def _pool_np(edges, cpe):
    n_joints = max(max(e) for e in edges) + 1
    degree = [0] * (n_joints + 1)
    for a, b in edges:
        degree[a] += 1
        degree[b] += 1
    seq_list = []

    def find_seq(j, seq):
        if degree[j] > 2 and j != 0:
            seq_list.append(seq)
            seq = []
        if degree[j] == 1:
            seq_list.append(seq)
            return
        for idx, e in enumerate(edges):
            if e[0] == j:
                find_seq(e[1], seq + [idx])

    find_seq(0, [])
    pooling_list = []
    for seq in seq_list:
        if len(seq) % 2 == 1:
            pooling_list.append([seq[0]])
            seq = seq[1:]
        for i in range(0, len(seq), 2):
            pooling_list.append([seq[i], seq[i + 1]])
    edge_num = len(edges) + 1
    pooling_list.append([edge_num - 1])
    w = np.zeros((len(pooling_list) * cpe, edge_num * cpe), dtype=np.float32)
    for i, pair in enumerate(pooling_list):
        for j in pair:
            w[i * cpe:(i + 1) * cpe, j * cpe:(j + 1) * cpe] += (
                np.eye(cpe, dtype=np.float32) / len(pair))
    return w


_NBL = _neighbor_list(_EDGES, _DIST)
_MASK = _mask_np(_NBL, _CH_IN, _CH_OUT)
_POOL = _pool_np(_EDGES, _CH_OUT)


def _enc_kernel(xt_ref, wt_ref, bias_ref, maskt_ref, pool_ref,
                out_ref, wc_ref, bc_ref):
    @pl.when(pl.program_id(0) == 0)
    def _fold():
        wmt = wt_ref[...] * maskt_ref[...]
        wc_ref[...] = lax.dot_general(
            pool_ref[...], wmt, (((1,), (1,)), ((), ())),
            preferred_element_type=jnp.float32)
        bc_ref[...] = lax.dot_general(
            pool_ref[...], bias_ref[...].reshape(-1, 1),
            (((1,), (0,)), ((), ())),
            preferred_element_type=jnp.float32)

    acc = lax.dot_general(
        wc_ref[...].astype(jnp.bfloat16), xt_ref[...].astype(jnp.bfloat16),
        (((1,), (0,)), ((), ())),
        preferred_element_type=jnp.float32)
    out_ref[...] = jnp.maximum(acc + bc_ref[...], 0.0)


@functools.partial(jax.jit, static_argnames=())
def kernel(input, weight, bias):
    B, in_ch = input.shape
    out_ch = _POOL.shape[0]
    maskt = jnp.asarray(_MASK.T)
    pool = jnp.asarray(_POOL)
    bias2 = bias

    n_mask = _MASK.shape[0]

    TB = 8192
    grid = (B // TB,)
    out_t = pl.pallas_call(
        _enc_kernel,
        grid=grid,
        in_specs=[
            pl.BlockSpec((in_ch, TB), lambda i: (0, i)),
            pl.BlockSpec((in_ch, n_mask), lambda i: (0, 0)),
            pl.BlockSpec((n_mask,), lambda i: (0,)),
            pl.BlockSpec((in_ch, n_mask), lambda i: (0, 0)),
            pl.BlockSpec((out_ch, n_mask), lambda i: (0, 0)),
        ],
        out_specs=pl.BlockSpec((out_ch, TB), lambda i: (0, i)),
        out_shape=jax.ShapeDtypeStruct((out_ch, B), jnp.float32),
        scratch_shapes=[
            pltpu.VMEM((out_ch, in_ch), jnp.float32),
            pltpu.VMEM((out_ch, 1), jnp.float32),
        ],
    )(input.T, weight.T, bias2, maskt, pool)

    return out_t.T[..., None]


# speedup vs baseline: 1.0715x; 1.0025x over previous
import functools

import numpy as np
import jax
import jax.numpy as jnp
from jax import lax
from jax.experimental import pallas as pl
from jax.experimental.pallas import tpu as pltpu

_EDGES = [[0, 1], [1, 2], [2, 3], [3, 4], [0, 5], [5, 6], [6, 7], [7, 8],
          [0, 9], [9, 10], [10, 11], [11, 12], [12, 13], [11, 14], [14, 15],
          [15, 16], [16, 17], [11, 18], [18, 19], [19, 20], [20, 21]]
_DIST = 1
_CH_IN = 3
_CH_OUT = 24


def _neighbor_list(edges, d):
    n = len(edges)
    inf = 1e9
    mat = np.full((n, n), inf)
    for i, a in enumerate(edges):
        for j, b in enumerate(edges):
            if set(a) & set(b):
                mat[i, j] = 1.0
    np.fill_diagonal(mat, 0.0)
    for k in range(n):
        mat = np.minimum(mat, mat[:, k:k + 1] + mat[k:k + 1, :])
    nbl = [[j for j in range(n) if mat[i, j] <= d] for i in range(n)]
    root_nb = list(nbl[0])
    for i in root_nb:
        nbl[i].append(n)
    root_nb.append(n)
    nbl.append(root_nb)
    return nbl


def _mask_np(nbl, cpj_in, cpj_out):
    n = len(nbl)
    mask = np.zeros((n * cpj_out, n * cpj_in), dtype=np.float32)
    for i, nbs in enumerate(nbl):
        for j in nbs:
            mask[i * cpj_out:(i + 1) * cpj_out, j * cpj_in:(j + 1) * cpj_in] = 1.0
    return mask


def _pool_np(edges, cpe):
    n_joints = max(max(e) for e in edges) + 1
    degree = [0] * (n_joints + 1)
    for a, b in edges:
        degree[a] += 1
        degree[b] += 1
    seq_list = []

    def find_seq(j, seq):
        if degree[j] > 2 and j != 0:
            seq_list.append(seq)
            seq = []
        if degree[j] == 1:
            seq_list.append(seq)
            return
        for idx, e in enumerate(edges):
            if e[0] == j:
                find_seq(e[1], seq + [idx])

    find_seq(0, [])
    pooling_list = []
    for seq in seq_list:
        if len(seq) % 2 == 1:
            pooling_list.append([seq[0]])
            seq = seq[1:]
        for i in range(0, len(seq), 2):
            pooling_list.append([seq[i], seq[i + 1]])
    edge_num = len(edges) + 1
    pooling_list.append([edge_num - 1])
    w = np.zeros((len(pooling_list) * cpe, edge_num * cpe), dtype=np.float32)
    for i, pair in enumerate(pooling_list):
        for j in pair:
            w[i * cpe:(i + 1) * cpe, j * cpe:(j + 1) * cpe] += (
                np.eye(cpe, dtype=np.float32) / len(pair))
    return w


_NBL = _neighbor_list(_EDGES, _DIST)
_MASK = _mask_np(_NBL, _CH_IN, _CH_OUT)
_POOL = _pool_np(_EDGES, _CH_OUT)


def _enc_kernel(xt_ref, wt_ref, bias_ref, maskt_ref, pool_ref,
                out_ref, wc_ref, bc_ref):
    @pl.when(pl.program_id(0) == 0)
    def _fold():
        wmt = wt_ref[...] * maskt_ref[...]
        wc_ref[...] = lax.dot_general(
            pool_ref[...], wmt, (((1,), (1,)), ((), ())),
            preferred_element_type=jnp.float32)
        bc_ref[...] = lax.dot_general(
            pool_ref[...], bias_ref[...].reshape(-1, 1),
            (((1,), (0,)), ((), ())),
            preferred_element_type=jnp.float32)

    acc = lax.dot_general(
        wc_ref[...].astype(jnp.bfloat16), xt_ref[...].astype(jnp.bfloat16),
        (((1,), (0,)), ((), ())),
        preferred_element_type=jnp.float32)
    out_ref[...] = jnp.maximum(acc + bc_ref[...], 0.0)


@functools.partial(jax.jit, static_argnames=())
def kernel(input, weight, bias):
    B, in_ch = input.shape
    out_ch = _POOL.shape[0]
    maskt = jnp.asarray(_MASK.T)
    pool = jnp.asarray(_POOL)
    bias2 = bias

    n_mask = _MASK.shape[0]

    TB = 4096
    grid = (B // TB,)
    out_t = pl.pallas_call(
        _enc_kernel,
        grid=grid,
        in_specs=[
            pl.BlockSpec((in_ch, TB), lambda i: (0, i)),
            pl.BlockSpec((in_ch, n_mask), lambda i: (0, 0)),
            pl.BlockSpec((n_mask,), lambda i: (0,)),
            pl.BlockSpec((in_ch, n_mask), lambda i: (0, 0)),
            pl.BlockSpec((out_ch, n_mask), lambda i: (0, 0)),
        ],
        out_specs=pl.BlockSpec((out_ch, TB), lambda i: (0, i)),
        out_shape=jax.ShapeDtypeStruct((out_ch, B), jnp.float32),
        scratch_shapes=[
            pltpu.VMEM((out_ch, in_ch), jnp.float32),
            pltpu.VMEM((out_ch, 1), jnp.float32),
        ],
    )(input.T, weight.T, bias2, maskt, pool)

    return out_t.T[..., None]


# speedup vs baseline: 1.0716x; 1.0001x over previous
import functools

import numpy as np
import jax
import jax.numpy as jnp
from jax import lax
from jax.experimental import pallas as pl
from jax.experimental.pallas import tpu as pltpu

_EDGES = [[0, 1], [1, 2], [2, 3], [3, 4], [0, 5], [5, 6], [6, 7], [7, 8],
          [0, 9], [9, 10], [10, 11], [11, 12], [12, 13], [11, 14], [14, 15],
          [15, 16], [16, 17], [11, 18], [18, 19], [19, 20], [20, 21]]
_DIST = 1
_CH_IN = 3
_CH_OUT = 24


def _neighbor_list(edges, d):
    n = len(edges)
    inf = 1e9
    mat = np.full((n, n), inf)
    for i, a in enumerate(edges):
        for j, b in enumerate(edges):
            if set(a) & set(b):
                mat[i, j] = 1.0
    np.fill_diagonal(mat, 0.0)
    for k in range(n):
        mat = np.minimum(mat, mat[:, k:k + 1] + mat[k:k + 1, :])
    nbl = [[j for j in range(n) if mat[i, j] <= d] for i in range(n)]
    root_nb = list(nbl[0])
    for i in root_nb:
        nbl[i].append(n)
    root_nb.append(n)
    nbl.append(root_nb)
    return nbl


def _mask_np(nbl, cpj_in, cpj_out):
    n = len(nbl)
    mask = np.zeros((n * cpj_out, n * cpj_in), dtype=np.float32)
    for i, nbs in enumerate(nbl):
        for j in nbs:
            mask[i * cpj_out:(i + 1) * cpj_out, j * cpj_in:(j + 1) * cpj_in] = 1.0
    return mask


def _pool_np(edges, cpe):
    n_joints = max(max(e) for e in edges) + 1
    degree = [0] * (n_joints + 1)
    for a, b in edges:
        degree[a] += 1
        degree[b] += 1
    seq_list = []

    def find_seq(j, seq):
        if degree[j] > 2 and j != 0:
            seq_list.append(seq)
            seq = []
        if degree[j] == 1:
            seq_list.append(seq)
            return
        for idx, e in enumerate(edges):
            if e[0] == j:
                find_seq(e[1], seq + [idx])

    find_seq(0, [])
    pooling_list = []
    for seq in seq_list:
        if len(seq) % 2 == 1:
            pooling_list.append([seq[0]])
            seq = seq[1:]
        for i in range(0, len(seq), 2):
            pooling_list.append([seq[i], seq[i + 1]])
    edge_num = len(edges) + 1
    pooling_list.append([edge_num - 1])
    w = np.zeros((len(pooling_list) * cpe, edge_num * cpe), dtype=np.float32)
    for i, pair in enumerate(pooling_list):
        for j in pair:
            w[i * cpe:(i + 1) * cpe, j * cpe:(j + 1) * cpe] += (
                np.eye(cpe, dtype=np.float32) / len(pair))
    return w


_NBL = _neighbor_list(_EDGES, _DIST)
_MASK = _mask_np(_NBL, _CH_IN, _CH_OUT)
_POOL = _pool_np(_EDGES, _CH_OUT)


def _enc_kernel(xt_ref, wt_ref, bias_ref, maskt_ref, pool_ref,
                out_ref, wc_ref, bc_ref):
    @pl.when(pl.program_id(0) == 0)
    def _fold():
        wmt = wt_ref[...] * maskt_ref[...]
        wc_ref[...] = lax.dot_general(
            pool_ref[...], wmt, (((1,), (1,)), ((), ())),
            preferred_element_type=jnp.float32)
        bc_ref[...] = lax.dot_general(
            pool_ref[...], bias_ref[...].reshape(-1, 1),
            (((1,), (0,)), ((), ())),
            preferred_element_type=jnp.float32)

    acc = lax.dot_general(
        wc_ref[...].astype(jnp.bfloat16), xt_ref[...].astype(jnp.bfloat16),
        (((1,), (0,)), ((), ())),
        preferred_element_type=jnp.float32)
    out_ref[...] = jnp.maximum(acc + bc_ref[...], 0.0)


@functools.partial(jax.jit, static_argnames=())
def kernel(input, weight, bias):
    B, in_ch = input.shape
    out_ch = _POOL.shape[0]
    maskt = jnp.asarray(_MASK.T)
    pool = jnp.asarray(_POOL)
    bias2 = bias

    n_mask = _MASK.shape[0]

    TB = 8192
    grid = (B // TB,)
    out_t = pl.pallas_call(
        _enc_kernel,
        grid=grid,
        in_specs=[
            pl.BlockSpec((in_ch, TB), lambda i: (0, i)),
            pl.BlockSpec((in_ch, n_mask), lambda i: (0, 0)),
            pl.BlockSpec((n_mask,), lambda i: (0,)),
            pl.BlockSpec((in_ch, n_mask), lambda i: (0, 0)),
            pl.BlockSpec((out_ch, n_mask), lambda i: (0, 0)),
        ],
        out_specs=pl.BlockSpec((out_ch, TB), lambda i: (0, i)),
        out_shape=jax.ShapeDtypeStruct((out_ch, B), jnp.float32),
        scratch_shapes=[
            pltpu.VMEM((out_ch, in_ch), jnp.float32),
            pltpu.VMEM((out_ch, 1), jnp.float32),
        ],
    )(input.T, weight.T, bias2, maskt, pool)

    return out_t.T[..., None]
